# baseline (speedup 1.0000x reference)
import functools

import jax
import jax.numpy as jnp
from jax import lax
from jax.experimental import pallas as pl
from jax.experimental.pallas import tpu as pltpu
from jax.experimental.pallas import tpu_sc as plsc

_B, _N, _C = 16, 1024, 128
_K = 16
_NODES = _B * _N
_NW = 32
_NPW = _NODES // _NW
_NCH = 8
_GCH = _NCH * _K


def _mm(a, b):
    return lax.dot_general(a, b, (((1,), (0,)), ((), ())),
                           preferred_element_type=jnp.float32,
                           precision=lax.Precision.HIGHEST)


def _stage1_body(pf_ref, bx_ref, pt_ref, wga_ref, wgb_ref, cg_ref,
                 wa_ref, wb_ref, c1_ref,
                 h_ref, idx_ref, u_ref, r_ref, d_ref):
    pf = pf_ref[...]
    bx = bx_ref[...]
    h = jnp.maximum(_mm(pf, wga_ref[...]) + _mm(bx, wgb_ref[...])
                    + cg_ref[...], 0.0)
    h_ref[...] = h
    u_ref[...] = _mm(h, wa_ref[...])
    r_ref[...] = _mm(h, wb_ref[...]) + c1_ref[...]
    pt = pt_ref[...]
    d = None
    for c in range(3):
        diff = bx[:, c:c + 1] - pt[c:c + 1, :]
        sq = diff * diff
        d = sq if d is None else d + sq
    ii = lax.broadcasted_iota(jnp.int32, (_N, _N), 0)
    jj = lax.broadcasted_iota(jnp.int32, (_N, _N), 1)
    d = jnp.where(ii == jj, d + 1e10, d)
    d_ref[...] = d
    base = pl.program_id(0) * _N
    for t in range(_K):
        dc = d_ref[...]
        rm = jnp.min(dc, axis=1, keepdims=True)
        am = jnp.min(jnp.where(dc == rm, jj, _N), axis=1, keepdims=True)
        idx_ref[:, t:t + 1] = am + base
        d_ref[...] = jnp.where(jj == am, 1e30, dc)


def _stage1(pf, bx, pt, wga, wgb, cg, wa, wb, c1):
    nb = pf.shape[0]
    nodes = nb * _N
    return pl.pallas_call(
        _stage1_body,
        grid=(nb,),
        in_specs=[
            pl.BlockSpec((None, _N, _C), lambda b: (b, 0, 0)),
            pl.BlockSpec((None, _N, 8), lambda b: (b, 0, 0)),
            pl.BlockSpec((None, 8, _N), lambda b: (b, 0, 0)),
            pl.BlockSpec((_C, _C), lambda b: (0, 0)),
            pl.BlockSpec((8, _C), lambda b: (0, 0)),
            pl.BlockSpec((1, _C), lambda b: (0, 0)),
            pl.BlockSpec((_C, _C), lambda b: (0, 0)),
            pl.BlockSpec((_C, _C), lambda b: (0, 0)),
            pl.BlockSpec((1, _C), lambda b: (0, 0)),
        ],
        out_specs=[
            pl.BlockSpec((_N, _C), lambda b: (b, 0)),
            pl.BlockSpec((_N, _K), lambda b: (b, 0)),
            pl.BlockSpec((_N, _C), lambda b: (b, 0)),
            pl.BlockSpec((_N, _C), lambda b: (b, 0)),
        ],
        out_shape=[
            jax.ShapeDtypeStruct((nodes, _C), jnp.float32),
            jax.ShapeDtypeStruct((nodes, _K), jnp.int32),
            jax.ShapeDtypeStruct((nodes, _C), jnp.float32),
            jax.ShapeDtypeStruct((nodes, _C), jnp.float32),
        ],
        scratch_shapes=[pltpu.VMEM((_N, _N), jnp.float32)],
    )(pf, bx, pt, wga, wgb, cg, wa, wb, c1)


def _gmax_sc(u, idx_flat):
    mesh = plsc.VectorSubcoreMesh(core_axis_name="c", subcore_axis_name="s")
    nodes = u.shape[0]
    npw = nodes // _NW
    nchunks = npw // _NCH

    @functools.partial(
        pl.kernel, mesh=mesh,
        out_type=jax.ShapeDtypeStruct((nodes, _C), jnp.float32),
        scratch_types=[
            pltpu.VMEM((npw * _K,), jnp.int32),
            pltpu.VMEM((_GCH, _C), jnp.float32),
            pltpu.VMEM((_GCH, _C), jnp.float32),
            pltpu.VMEM((_NCH, _C), jnp.float32),
            pltpu.SemaphoreType.DMA,
            pltpu.SemaphoreType.DMA,
        ],
    )
    def k(u_hbm, idx_hbm, out_hbm, idx_v, rows0, rows1, m_v, sem0, sem1):
        wid = lax.axis_index("s") * 2 + lax.axis_index("c")
        nbase = wid * npw
        rows = (rows0, rows1)
        sems = (sem0, sem1)
        pltpu.sync_copy(idx_hbm.at[pl.ds(nbase * _K, npw * _K)], idx_v)
        pltpu.async_copy(u_hbm.at[idx_v.at[pl.ds(0, _GCH)]], rows0, sem0)

        def outer(oi, carry):
            for b in range(2):
                ci = oi * 2 + b
                nxt = ci + 1

                @pl.when(nxt < nchunks)
                def _():
                    pltpu.async_copy(
                        u_hbm.at[idx_v.at[pl.ds(nxt * _GCH, _GCH)]],
                        rows[1 - b], sems[1 - b])

                pltpu.make_async_copy(
                    u_hbm.at[pl.ds(0, _GCH)], rows[b], sems[b]).wait()
                for n in range(_NCH):
                    for c in range(_C // 16):
                        sl = pl.ds(c * 16, 16)
                        acc = rows[b][n * _K, sl]
                        for r2 in range(1, _K):
                            acc = jnp.maximum(acc, rows[b][n * _K + r2, sl])
                        m_v[n, sl] = acc
                pltpu.sync_copy(m_v,
                                out_hbm.at[pl.ds(nbase + ci * _NCH, _NCH)])
            return carry

        lax.fori_loop(0, nchunks // 2, outer, 0)

    return k(u, idx_flat)


def _mid_body(m_ref, r_ref, a_ref, wa_ref, wb_ref, c_ref,
              x_ref, u_ref, r2_ref):
    x = jnp.maximum(a_ref[...] * m_ref[...] + r_ref[...], 0.0)
    x_ref[...] = x
    u_ref[...] = _mm(x, wa_ref[...])
    r2_ref[...] = _mm(x, wb_ref[...]) + c_ref[...]


def _mid(m1, r1, a1, wa2, wb2, c2):
    nodes = m1.shape[0]
    return pl.pallas_call(
        _mid_body,
        grid=(nodes // _N,),
        in_specs=[
            pl.BlockSpec((_N, _C), lambda b: (b, 0)),
            pl.BlockSpec((_N, _C), lambda b: (b, 0)),
            pl.BlockSpec((1, _C), lambda b: (0, 0)),
            pl.BlockSpec((_C, _C), lambda b: (0, 0)),
            pl.BlockSpec((_C, _C), lambda b: (0, 0)),
            pl.BlockSpec((1, _C), lambda b: (0, 0)),
        ],
        out_specs=[
            pl.BlockSpec((_N, _C), lambda b: (b, 0)),
            pl.BlockSpec((_N, _C), lambda b: (b, 0)),
            pl.BlockSpec((_N, _C), lambda b: (b, 0)),
        ],
        out_shape=[
            jax.ShapeDtypeStruct((nodes, _C), jnp.float32),
            jax.ShapeDtypeStruct((nodes, _C), jnp.float32),
            jax.ShapeDtypeStruct((nodes, _C), jnp.float32),
        ],
    )(m1, r1, a1, wa2, wb2, c2)


def _fin_body(h_ref, x1_ref, m_ref, r_ref, a_ref, o_ref):
    o_ref[:, 0:_C] = h_ref[...]
    o_ref[:, _C:2 * _C] = x1_ref[...]
    o_ref[:, 2 * _C:3 * _C] = jnp.maximum(
        a_ref[...] * m_ref[...] + r_ref[...], 0.0)


def _fin(h, x1, m2, r2, a2):
    nodes = h.shape[0]
    return pl.pallas_call(
        _fin_body,
        grid=(nodes // _N,),
        in_specs=[
            pl.BlockSpec((_N, _C), lambda b: (b, 0)),
            pl.BlockSpec((_N, _C), lambda b: (b, 0)),
            pl.BlockSpec((_N, _C), lambda b: (b, 0)),
            pl.BlockSpec((_N, _C), lambda b: (b, 0)),
            pl.BlockSpec((1, _C), lambda b: (0, 0)),
        ],
        out_specs=pl.BlockSpec((_N, 3 * _C), lambda b: (b, 0)),
        out_shape=jax.ShapeDtypeStruct((nodes, 3 * _C), jnp.float32),
    )(h, x1, m2, r2, a2)


def kernel(pooled_features, rois, roi_labels, Wg, bg, gg, gb,
           W1, b1, g1, be1, W2, b2, g2, be2):
    f32 = jnp.float32
    inv = (1.0 / jnp.sqrt(jnp.asarray(1.0 + 1e-5, f32))).astype(f32)
    bx = jnp.pad(rois, ((0, 0), (0, 0), (0, 1)))
    pt = jnp.pad(jnp.swapaxes(rois, 1, 2), ((0, 0), (0, 1), (0, 0)))

    sg = gg * inv
    wgp = Wg * sg[None, :]
    wga = wgp[:_C]
    wgb = jnp.pad(wgp[_C:], ((0, 1), (0, 0)))
    cg = (bg * sg + gb)[None, :]

    def fold(W, b, g, be):
        s = g * inv
        sign = jnp.where(s >= 0, f32(1.0), f32(-1.0))
        wa = W[:_C] * sign[None, :]
        wb = (W[_C:] - W[:_C]) * s[None, :]
        cc = (b * s + be)[None, :]
        aa = jnp.abs(s)[None, :]
        return wa, wb, cc, aa

    wa1, wb1, c1, a1 = fold(W1, b1, g1, be1)
    wa2, wb2, c2, a2 = fold(W2, b2, g2, be2)

    halves = []
    hb = _B // 4
    for lo in (0, hb, 2 * hb, 3 * hb):
        h, idxg, u1, r1 = _stage1(pooled_features[lo:lo + hb],
                                  bx[lo:lo + hb], pt[lo:lo + hb],
                                  wga, wgb, cg, wa1, wb1, c1)
        idx_flat = idxg.reshape(-1)
        m1 = _gmax_sc(u1, idx_flat)
        x1, u2, r2 = _mid(m1, r1, a1, wa2, wb2, c2)
        m2 = _gmax_sc(u2, idx_flat)
        halves.append(_fin(h, x1, m2, r2, a2))
    return jnp.concatenate(halves, axis=0)


# speedup vs baseline: 1.0238x; 1.0238x over previous
import functools

import jax
import jax.numpy as jnp
from jax import lax
from jax.experimental import pallas as pl
from jax.experimental.pallas import tpu as pltpu
from jax.experimental.pallas import tpu_sc as plsc

_B, _N, _C = 16, 1024, 128
_K = 16
_NODES = _B * _N
_NW = 32
_NPW = _NODES // _NW
_NCH = 8
_GCH = _NCH * _K


def _mm(a, b):
    return lax.dot_general(a, b, (((1,), (0,)), ((), ())),
                           preferred_element_type=jnp.float32,
                           precision=lax.Precision.HIGHEST)


def _stage1_body(pf_ref, bx_ref, pt_ref, wga_ref, wgb_ref, cg_ref,
                 wa_ref, wb_ref, c1_ref,
                 h_ref, idx_ref, u_ref, r_ref, d_ref):
    pf = pf_ref[...]
    bx = bx_ref[...]
    h = jnp.maximum(_mm(pf, wga_ref[...]) + _mm(bx, wgb_ref[...])
                    + cg_ref[...], 0.0)
    h_ref[...] = h
    u_ref[...] = _mm(h, wa_ref[...])
    r_ref[...] = _mm(h, wb_ref[...]) + c1_ref[...]
    pt = pt_ref[...]
    d = None
    for c in range(3):
        diff = bx[:, c:c + 1] - pt[c:c + 1, :]
        sq = diff * diff
        d = sq if d is None else d + sq
    ii = lax.broadcasted_iota(jnp.int32, (_N, _N), 0)
    jj = lax.broadcasted_iota(jnp.int32, (_N, _N), 1)
    d = jnp.where(ii == jj, d + 1e10, d)
    d_ref[...] = d
    base = pl.program_id(0) * _N
    for t in range(_K):
        dc = d_ref[...]
        rm = jnp.min(dc, axis=1, keepdims=True)
        am = jnp.min(jnp.where(dc == rm, jj, _N), axis=1, keepdims=True)
        idx_ref[:, t:t + 1] = am + base
        d_ref[...] = jnp.where(jj == am, 1e30, dc)


def _stage1(pf, bx, pt, wga, wgb, cg, wa, wb, c1):
    nb = pf.shape[0]
    nodes = nb * _N
    return pl.pallas_call(
        _stage1_body,
        grid=(nb,),
        in_specs=[
            pl.BlockSpec((None, _N, _C), lambda b: (b, 0, 0)),
            pl.BlockSpec((None, _N, 8), lambda b: (b, 0, 0)),
            pl.BlockSpec((None, 8, _N), lambda b: (b, 0, 0)),
            pl.BlockSpec((_C, _C), lambda b: (0, 0)),
            pl.BlockSpec((8, _C), lambda b: (0, 0)),
            pl.BlockSpec((1, _C), lambda b: (0, 0)),
            pl.BlockSpec((_C, _C), lambda b: (0, 0)),
            pl.BlockSpec((_C, _C), lambda b: (0, 0)),
            pl.BlockSpec((1, _C), lambda b: (0, 0)),
        ],
        out_specs=[
            pl.BlockSpec((_N, _C), lambda b: (b, 0)),
            pl.BlockSpec((_N, _K), lambda b: (b, 0)),
            pl.BlockSpec((_N, _C), lambda b: (b, 0)),
            pl.BlockSpec((_N, _C), lambda b: (b, 0)),
        ],
        out_shape=[
            jax.ShapeDtypeStruct((nodes, _C), jnp.float32),
            jax.ShapeDtypeStruct((nodes, _K), jnp.int32),
            jax.ShapeDtypeStruct((nodes, _C), jnp.float32),
            jax.ShapeDtypeStruct((nodes, _C), jnp.float32),
        ],
        scratch_shapes=[pltpu.VMEM((_N, _N), jnp.float32)],
    )(pf, bx, pt, wga, wgb, cg, wa, wb, c1)


def _gmax_sc(u, idx_flat):
    mesh = plsc.VectorSubcoreMesh(core_axis_name="c", subcore_axis_name="s")
    nodes = u.shape[0]
    npw = nodes // _NW
    nchunks = npw // _NCH

    @functools.partial(
        pl.kernel, mesh=mesh,
        out_type=jax.ShapeDtypeStruct((nodes, _C), jnp.float32),
        scratch_types=[
            pltpu.VMEM((npw * _K,), jnp.int32),
            pltpu.VMEM((_GCH, _C), jnp.float32),
            pltpu.VMEM((_GCH, _C), jnp.float32),
            pltpu.VMEM((_NCH, _C), jnp.float32),
            pltpu.SemaphoreType.DMA,
            pltpu.SemaphoreType.DMA,
        ],
    )
    def k(u_hbm, idx_hbm, out_hbm, idx_v, rows0, rows1, m_v, sem0, sem1):
        wid = lax.axis_index("s") * 2 + lax.axis_index("c")
        nbase = wid * npw
        rows = (rows0, rows1)
        sems = (sem0, sem1)
        pltpu.sync_copy(idx_hbm.at[pl.ds(nbase * _K, npw * _K)], idx_v)
        pltpu.async_copy(u_hbm.at[idx_v.at[pl.ds(0, _GCH)]], rows0, sem0)

        def outer(oi, carry):
            for b in range(2):
                ci = oi * 2 + b
                nxt = ci + 1

                @pl.when(nxt < nchunks)
                def _():
                    pltpu.async_copy(
                        u_hbm.at[idx_v.at[pl.ds(nxt * _GCH, _GCH)]],
                        rows[1 - b], sems[1 - b])

                pltpu.make_async_copy(
                    u_hbm.at[pl.ds(0, _GCH)], rows[b], sems[b]).wait()
                for n in range(_NCH):
                    for c in range(_C // 16):
                        sl = pl.ds(c * 16, 16)
                        acc = rows[b][n * _K, sl]
                        for r2 in range(1, _K):
                            acc = jnp.maximum(acc, rows[b][n * _K + r2, sl])
                        m_v[n, sl] = acc
                pltpu.sync_copy(m_v,
                                out_hbm.at[pl.ds(nbase + ci * _NCH, _NCH)])
            return carry

        lax.fori_loop(0, nchunks // 2, outer, 0)

    return k(u, idx_flat)


def _mid_body(m_ref, r_ref, a_ref, wa_ref, wb_ref, c_ref,
              x_ref, u_ref, r2_ref):
    x = jnp.maximum(a_ref[...] * m_ref[...] + r_ref[...], 0.0)
    x_ref[...] = x
    u_ref[...] = _mm(x, wa_ref[...])
    r2_ref[...] = _mm(x, wb_ref[...]) + c_ref[...]


def _mid(m1, r1, a1, wa2, wb2, c2):
    nodes = m1.shape[0]
    return pl.pallas_call(
        _mid_body,
        grid=(nodes // _N,),
        in_specs=[
            pl.BlockSpec((_N, _C), lambda b: (b, 0)),
            pl.BlockSpec((_N, _C), lambda b: (b, 0)),
            pl.BlockSpec((1, _C), lambda b: (0, 0)),
            pl.BlockSpec((_C, _C), lambda b: (0, 0)),
            pl.BlockSpec((_C, _C), lambda b: (0, 0)),
            pl.BlockSpec((1, _C), lambda b: (0, 0)),
        ],
        out_specs=[
            pl.BlockSpec((_N, _C), lambda b: (b, 0)),
            pl.BlockSpec((_N, _C), lambda b: (b, 0)),
            pl.BlockSpec((_N, _C), lambda b: (b, 0)),
        ],
        out_shape=[
            jax.ShapeDtypeStruct((nodes, _C), jnp.float32),
            jax.ShapeDtypeStruct((nodes, _C), jnp.float32),
            jax.ShapeDtypeStruct((nodes, _C), jnp.float32),
        ],
    )(m1, r1, a1, wa2, wb2, c2)


def _fin_body(h_ref, x1_ref, m_ref, r_ref, a_ref, o_ref):
    o_ref[:, 0:_C] = h_ref[...]
    o_ref[:, _C:2 * _C] = x1_ref[...]
    o_ref[:, 2 * _C:3 * _C] = jnp.maximum(
        a_ref[...] * m_ref[...] + r_ref[...], 0.0)


def _fin(h, x1, m2, r2, a2):
    nodes = h.shape[0]
    return pl.pallas_call(
        _fin_body,
        grid=(nodes // _N,),
        in_specs=[
            pl.BlockSpec((_N, _C), lambda b: (b, 0)),
            pl.BlockSpec((_N, _C), lambda b: (b, 0)),
            pl.BlockSpec((_N, _C), lambda b: (b, 0)),
            pl.BlockSpec((_N, _C), lambda b: (b, 0)),
            pl.BlockSpec((1, _C), lambda b: (0, 0)),
        ],
        out_specs=pl.BlockSpec((_N, 3 * _C), lambda b: (b, 0)),
        out_shape=jax.ShapeDtypeStruct((nodes, 3 * _C), jnp.float32),
    )(h, x1, m2, r2, a2)


def kernel(pooled_features, rois, roi_labels, Wg, bg, gg, gb,
           W1, b1, g1, be1, W2, b2, g2, be2):
    f32 = jnp.float32
    inv = (1.0 / jnp.sqrt(jnp.asarray(1.0 + 1e-5, f32))).astype(f32)
    bx = jnp.pad(rois, ((0, 0), (0, 0), (0, 1)))
    pt = jnp.pad(jnp.swapaxes(rois, 1, 2), ((0, 0), (0, 1), (0, 0)))

    sg = gg * inv
    wgp = Wg * sg[None, :]
    wga = wgp[:_C]
    wgb = jnp.pad(wgp[_C:], ((0, 1), (0, 0)))
    cg = (bg * sg + gb)[None, :]

    def fold(W, b, g, be):
        s = g * inv
        sign = jnp.where(s >= 0, f32(1.0), f32(-1.0))
        wa = W[:_C] * sign[None, :]
        wb = (W[_C:] - W[:_C]) * s[None, :]
        cc = (b * s + be)[None, :]
        aa = jnp.abs(s)[None, :]
        return wa, wb, cc, aa

    wa1, wb1, c1, a1 = fold(W1, b1, g1, be1)
    wa2, wb2, c2, a2 = fold(W2, b2, g2, be2)

    halves = []
    hb = _B // 2
    for lo in (0, hb):
        h, idxg, u1, r1 = _stage1(pooled_features[lo:lo + hb],
                                  bx[lo:lo + hb], pt[lo:lo + hb],
                                  wga, wgb, cg, wa1, wb1, c1)
        idx_flat = idxg.reshape(-1)
        m1 = _gmax_sc(u1, idx_flat)
        x1, u2, r2 = _mid(m1, r1, a1, wa2, wb2, c2)
        m2 = _gmax_sc(u2, idx_flat)
        halves.append(_fin(h, x1, m2, r2, a2))
    return jnp.concatenate(halves, axis=0)
